# Initial kernel scaffold; baseline (speedup 1.0000x reference)
import jax
import jax.numpy as jnp
from jax.experimental import pallas as pl


def kernel(inputs, env_map):
    raise NotImplementedError("write your pallas kernel here")


# speedup vs baseline: 57.2208x; 57.2208x over previous
import functools

import jax
import jax.numpy as jnp
from jax import lax
from jax.experimental import pallas as pl
from jax.experimental.pallas import tpu as pltpu
from jax.experimental.pallas import tpu_sc as plsc

_NC = 2
_NS = 16
_NW = _NC * _NS
_L = 16

_ATAN_CO = (
    2.766283480e-03, -1.573124904e-02, 4.213762345e-02, -7.456854814e-02,
    1.061837063e-01, -1.419779779e-01, 1.999187203e-01, -3.333303671e-01,
    9.999999818e-01,
)

_PI = 3.14159265358979
_HALF_PI = _PI / 2.0


def _atan01(t):
    u = t * t
    r = jnp.float32(_ATAN_CO[0])
    for c in _ATAN_CO[1:]:
        r = r * u + jnp.float32(c)
    return r * t


def _atan2(a, b):
    aa = jnp.abs(a)
    ab = jnp.abs(b)
    mx = jnp.maximum(jnp.maximum(aa, ab), jnp.float32(1e-30))
    mn = jnp.minimum(aa, ab)
    r = _atan01(mn / mx)
    r = jnp.where(aa > ab, jnp.float32(_HALF_PI) - r, r)
    r = jnp.where(b < 0, jnp.float32(_PI) - r, r)
    return jnp.where(a < 0, -r, r)


def _sqrt_nn(u):
    u = jnp.maximum(u, jnp.float32(1e-30))
    i = lax.bitcast_convert_type(u, jnp.int32)
    i = jnp.int32(0x5F3759DF) - lax.shift_right_logical(i, jnp.int32(1))
    h = lax.bitcast_convert_type(i, jnp.float32)
    for _ in range(3):
        h = h * (jnp.float32(1.5) - jnp.float32(0.5) * u * h * h)
    return u * h


def _sc_body(x_hbm, y_hbm, z_hbm, env_hbm, r_hbm, g_hbm, b_hbm,
             xv, yv, zv, tbl, rv, gv, bv, npw):
    wid = lax.axis_index("s") * _NC + lax.axis_index("c")
    base = wid * npw
    pltpu.sync_copy(env_hbm, tbl)
    pltpu.sync_copy(x_hbm.at[pl.ds(base, npw)], xv)
    pltpu.sync_copy(y_hbm.at[pl.ds(base, npw)], yv)
    pltpu.sync_copy(z_hbm.at[pl.ds(base, npw)], zv)

    def body(i, carry):
        sl = pl.ds(i * _L, _L)
        x = xv[sl]
        y = yv[sl]
        z = zv[sl]

        ac = _atan2(_sqrt_nn((jnp.float32(1.0) - y) * (jnp.float32(1.0) + y)), y)
        at = _atan2(x, -z)
        ix = ac * jnp.float32(32.0 / _PI) - jnp.float32(0.5)
        iy = at * jnp.float32(8.0 / _PI) + jnp.float32(7.5)

        ix0 = (ix + jnp.float32(1.0)).astype(jnp.int32) - 1
        iy0 = (iy + jnp.float32(1.0)).astype(jnp.int32) - 1
        wx1 = ix - ix0.astype(jnp.float32)
        wy1 = iy - iy0.astype(jnp.float32)
        wx0 = jnp.float32(1.0) - wx1
        wy0 = jnp.float32(1.0) - wy1
        ix1 = ix0 + 1
        iy1 = iy0 + 1

        one = jnp.float32(1.0)
        zero = jnp.float32(0.0)
        vx0 = jnp.where((ix0 >= 0) & (ix0 <= 31), one, zero)
        vx1 = jnp.where((ix1 >= 0) & (ix1 <= 31), one, zero)
        vy0 = jnp.where((iy0 >= 0) & (iy0 <= 15), one, zero)
        vy1 = jnp.where((iy1 >= 0) & (iy1 <= 15), one, zero)
        ix0c = jnp.minimum(jnp.maximum(ix0, 0), 31)
        ix1c = jnp.minimum(jnp.maximum(ix1, 0), 31)
        iy0c = jnp.minimum(jnp.maximum(iy0, 0), 15)
        iy1c = jnp.minimum(jnp.maximum(iy1, 0), 15)
        w00 = wy0 * wx0 * (vy0 * vx0)
        w01 = wy0 * wx1 * (vy0 * vx1)
        w10 = wy1 * wx0 * (vy1 * vx0)
        w11 = wy1 * wx1 * (vy1 * vx1)
        row0 = iy0c * 32
        row1 = iy1c * 32
        i00 = row0 + ix0c
        i01 = row0 + ix1c
        i10 = row1 + ix0c
        i11 = row1 + ix1c

        for out_ref, coff in ((rv, 0), (gv, 512), (bv, 1024)):
            c = jnp.int32(coff)
            s = (w00 * plsc.load_gather(tbl, [i00 + c])
                 + w01 * plsc.load_gather(tbl, [i01 + c])
                 + w10 * plsc.load_gather(tbl, [i10 + c])
                 + w11 * plsc.load_gather(tbl, [i11 + c]))
            out_ref[sl] = jnp.exp(s)
        return carry

    lax.fori_loop(0, npw // _L, body, 0)
    pltpu.sync_copy(rv, r_hbm.at[pl.ds(base, npw)])
    pltpu.sync_copy(gv, g_hbm.at[pl.ds(base, npw)])
    pltpu.sync_copy(bv, b_hbm.at[pl.ds(base, npw)])


def kernel(inputs, env_map):
    n = inputs.shape[0]
    npw = n // _NW
    xs = inputs[:, 0]
    ys = inputs[:, 1]
    zs = inputs[:, 2]
    env_flat = env_map.reshape(-1)

    mesh = plsc.VectorSubcoreMesh(
        core_axis_name="c", subcore_axis_name="s",
        num_cores=_NC, num_subcores=_NS)
    out_t = jax.ShapeDtypeStruct((n,), jnp.float32)
    sc_call = pl.kernel(
        functools.partial(_sc_body, npw=npw),
        out_type=(out_t, out_t, out_t),
        mesh=mesh,
        compiler_params=pltpu.CompilerParams(needs_layout_passes=False),
        scratch_types=(
            pltpu.VMEM((npw,), jnp.float32),
            pltpu.VMEM((npw,), jnp.float32),
            pltpu.VMEM((npw,), jnp.float32),
            pltpu.VMEM((1536,), jnp.float32),
            pltpu.VMEM((npw,), jnp.float32),
            pltpu.VMEM((npw,), jnp.float32),
            pltpu.VMEM((npw,), jnp.float32),
        ),
    )
    r, g, b = sc_call(xs, ys, zs, env_flat)
    return jnp.stack([r, g, b], axis=-1)
